# baseline (speedup 1.0000x reference)
import functools

import jax
import jax.numpy as jnp
from jax import lax
from jax.experimental import pallas as pl
from jax.experimental.pallas import tpu as pltpu
from jax.experimental.pallas import tpu_sc as plsc

_B = 16384
_L = 64
_GLOBAL_BIAS = 3.5

_INFO = plsc.get_sparse_core_info()
_NC = _INFO.num_cores
_NS = _INFO.num_subcores
_LANES = _INFO.num_lanes
_NW = _NC * _NS
_BPW = _B // _NW
_HALF = _BPW // 2
_HGROUPS = _HALF // _LANES
_C = 128


def _mf_kernel(users_hbm, items_hbm, ue_hbm, ie_hbm, ub_hbm, ib_hbm,
               out_hbm, idx_u, idx_i, idxp_u, idxp_i, ue_v, ie_v,
               ub_v, ib_v, out_v, sem_u, sem_i, sem_ub, sem_ib):
    wid = lax.axis_index("s") * _NC + lax.axis_index("c")
    base = wid * _BPW

    pltpu.sync_copy(users_hbm.at[pl.ds(base, _BPW)], idx_u)
    pltpu.sync_copy(items_hbm.at[pl.ds(base, _BPW)], idx_i)

    def pair_body(k, carry):
        s = pl.ds(k * _LANES, _LANES)
        idxp_u[s] = lax.shift_right_logical(idx_u[s], 1)
        idxp_i[s] = lax.shift_right_logical(idx_i[s], 1)
        return carry

    lax.fori_loop(0, _BPW // _LANES, pair_body, 0)

    bias_copies = []
    for c in range(_BPW // _C):
        s = pl.ds(c * _C, _C)
        bias_copies.append(pltpu.async_copy(
            ub_hbm.at[idx_u.at[s]], ub_v.at[s], sem_ub))
        bias_copies.append(pltpu.async_copy(
            ib_hbm.at[idx_i.at[s]], ib_v.at[s], sem_ib))

    for h in range(2):
        hbase = h * _HALF
        copies = []
        for c in range(_HALF // _C):
            s = pl.ds(hbase + c * _C, _C)
            d = pl.ds(c * _C, _C)
            copies.append(pltpu.async_copy(
                ue_hbm.at[idxp_u.at[s]], ue_v.at[d, :], sem_u))
            copies.append(pltpu.async_copy(
                ie_hbm.at[idxp_i.at[s]], ie_v.at[d, :], sem_i))
        for cp in copies:
            cp.wait()

        def group_body(g, carry):
            sl = pl.ds(hbase + g * _LANES, _LANES)
            rows = g * _LANES + lax.iota(jnp.int32, _LANES)
            u_half = jnp.bitwise_and(idx_u[sl], 1) * _L
            i_half = jnp.bitwise_and(idx_i[sl], 1) * _L
            acc = jnp.zeros((_LANES,), jnp.float32)
            for j in range(_L):
                u = plsc.load_gather(ue_v, [rows, u_half + j])
                v = plsc.load_gather(ie_v, [rows, i_half + j])
                acc = acc + u * v
            out_v[sl] = acc + _GLOBAL_BIAS
            return carry

        lax.fori_loop(0, _HGROUPS, group_body, 0)

    for cp in bias_copies:
        cp.wait()

    def bias_body(g, carry):
        s = pl.ds(g * _LANES, _LANES)
        out_v[s] = out_v[s] + ub_v[s] + ib_v[s]
        return carry

    lax.fori_loop(0, _BPW // _LANES, bias_body, 0)

    pltpu.sync_copy(out_v, out_hbm.at[pl.ds(base, _BPW)])


@functools.partial(
    pl.kernel,
    mesh=plsc.VectorSubcoreMesh(core_axis_name="c", subcore_axis_name="s"),
    out_type=jax.ShapeDtypeStruct((_B,), jnp.float32),
    compiler_params=pltpu.CompilerParams(needs_layout_passes=False),
    scratch_types=[
        pltpu.VMEM((_BPW,), jnp.int32),
        pltpu.VMEM((_BPW,), jnp.int32),
        pltpu.VMEM((_BPW,), jnp.int32),
        pltpu.VMEM((_BPW,), jnp.int32),
        pltpu.VMEM((_HALF, 2 * _L), jnp.float32),
        pltpu.VMEM((_HALF, 2 * _L), jnp.float32),
        pltpu.VMEM((_BPW,), jnp.float32),
        pltpu.VMEM((_BPW,), jnp.float32),
        pltpu.VMEM((_BPW,), jnp.float32),
        pltpu.SemaphoreType.DMA,
        pltpu.SemaphoreType.DMA,
        pltpu.SemaphoreType.DMA,
        pltpu.SemaphoreType.DMA,
    ],
)
def _mf_sc(users, items, ue, ie, ub, ib, out, *scratch):
    _mf_kernel(users, items, ue, ie, ub, ib, out, *scratch)


def kernel(users, items, uEmbed, itemEmbed, uBias, itemBias):
    n_u, l = uEmbed.shape
    n_i, _ = itemEmbed.shape
    score = _mf_sc(users.astype(jnp.int32), items.astype(jnp.int32),
                   uEmbed.reshape(n_u // 2, 2 * l),
                   itemEmbed.reshape(n_i // 2, 2 * l),
                   uBias.reshape(-1), itemBias.reshape(-1))
    return score.reshape(_B, 1)


# speedup vs baseline: 2.1246x; 2.1246x over previous
import functools

import jax
import jax.numpy as jnp
from jax import lax
from jax.experimental import pallas as pl
from jax.experimental.pallas import tpu as pltpu
from jax.experimental.pallas import tpu_sc as plsc

_B = 16384
_L = 64
_GLOBAL_BIAS = 3.5

_INFO = plsc.get_sparse_core_info()
_NC = _INFO.num_cores
_NS = _INFO.num_subcores
_LANES = _INFO.num_lanes
_NW = _NC * _NS
_BPW = _B // _NW
_CH = 32
_NCH = _BPW // _CH
_CGROUPS = _CH // _LANES
_C = 128


def _mf_kernel(users_hbm, items_hbm, ue3_hbm, ie3_hbm, ub_hbm, ib_hbm,
               out_hbm, idx_u, idx_i,
               ublk, iblk, ub_v, ib_v, out_v, sem_u, sem_i, sem_ub, sem_ib):
    wid = lax.axis_index("s") * _NC + lax.axis_index("c")
    base = wid * _BPW

    pltpu.sync_copy(users_hbm.at[pl.ds(base, _BPW)], idx_u)
    pltpu.sync_copy(items_hbm.at[pl.ds(base, _BPW)], idx_i)

    bias_copies = []
    for c in range(_BPW // _C):
        s = pl.ds(c * _C, _C)
        bias_copies.append(pltpu.async_copy(
            ub_hbm.at[idx_u.at[s]], ub_v.at[s], sem_ub))
        bias_copies.append(pltpu.async_copy(
            ib_hbm.at[idx_i.at[s]], ib_v.at[s], sem_ib))

    def chunk_body(c, carry):
        cbase = c * _CH

        def fire_body(g, carry2):
            s = pl.ds(cbase + g * _LANES, _LANES)
            vq_u = lax.shift_right_logical(idx_u[s], 3)
            vq_i = lax.shift_right_logical(idx_i[s], 3)
            for k in range(_LANES):
                r = g * _LANES + k
                pltpu.async_copy(ue3_hbm.at[vq_u[k]], ublk.at[r], sem_u)
                pltpu.async_copy(ie3_hbm.at[vq_i[k]], iblk.at[r], sem_i)
            return carry2

        lax.fori_loop(0, _CGROUPS, fire_body, 0)
        pltpu.make_async_copy(
            ue3_hbm.at[pl.ds(0, _CH)], ublk, sem_u).wait()
        pltpu.make_async_copy(
            ie3_hbm.at[pl.ds(0, _CH)], iblk, sem_i).wait()

        def group_body(g, carry3):
            sl = pl.ds(cbase + g * _LANES, _LANES)
            rloc = g * _LANES + lax.iota(jnp.int32, _LANES)
            usub = jnp.bitwise_and(idx_u[sl], 7)
            isub = jnp.bitwise_and(idx_i[sl], 7)
            acc = jnp.zeros((_LANES,), jnp.float32)
            for j in range(_L):
                jv = jnp.full((_LANES,), j, jnp.int32)
                u = plsc.load_gather(ublk, [rloc, usub, jv])
                v = plsc.load_gather(iblk, [rloc, isub, jv])
                acc = acc + u * v
            out_v[sl] = acc + _GLOBAL_BIAS
            return carry3

        lax.fori_loop(0, _CGROUPS, group_body, 0)
        return carry

    lax.fori_loop(0, _NCH, chunk_body, 0)

    for cp in bias_copies:
        cp.wait()

    def bias_body(g, carry):
        s = pl.ds(g * _LANES, _LANES)
        out_v[s] = out_v[s] + ub_v[s] + ib_v[s]
        return carry

    lax.fori_loop(0, _BPW // _LANES, bias_body, 0)

    pltpu.sync_copy(out_v, out_hbm.at[pl.ds(base, _BPW)])


@functools.partial(
    pl.kernel,
    mesh=plsc.VectorSubcoreMesh(core_axis_name="c", subcore_axis_name="s"),
    out_type=jax.ShapeDtypeStruct((_B,), jnp.float32),
    compiler_params=pltpu.CompilerParams(needs_layout_passes=False),
    scratch_types=[
        pltpu.VMEM((_BPW,), jnp.int32),
        pltpu.VMEM((_BPW,), jnp.int32),
        pltpu.VMEM((_CH, 8, _L), jnp.float32),
        pltpu.VMEM((_CH, 8, _L), jnp.float32),
        pltpu.VMEM((_BPW,), jnp.float32),
        pltpu.VMEM((_BPW,), jnp.float32),
        pltpu.VMEM((_BPW,), jnp.float32),
        pltpu.SemaphoreType.DMA,
        pltpu.SemaphoreType.DMA,
        pltpu.SemaphoreType.DMA,
        pltpu.SemaphoreType.DMA,
    ],
)
def _mf_sc(users, items, ue3, ie3, ub, ib, out, *scratch):
    _mf_kernel(users, items, ue3, ie3, ub, ib, out, *scratch)


def kernel(users, items, uEmbed, itemEmbed, uBias, itemBias):
    n_u, l = uEmbed.shape
    n_i, _ = itemEmbed.shape
    score = _mf_sc(users.astype(jnp.int32), items.astype(jnp.int32),
                   uEmbed.reshape(n_u // 8, 8, l),
                   itemEmbed.reshape(n_i // 8, 8, l),
                   uBias.reshape(-1), itemBias.reshape(-1))
    return score.reshape(_B, 1)


# speedup vs baseline: 2.2594x; 1.0634x over previous
import functools

import jax
import jax.numpy as jnp
from jax import lax
from jax.experimental import pallas as pl
from jax.experimental.pallas import tpu as pltpu
from jax.experimental.pallas import tpu_sc as plsc

_B = 16384
_L = 64
_GLOBAL_BIAS = 3.5

_INFO = plsc.get_sparse_core_info()
_NC = _INFO.num_cores
_NS = _INFO.num_subcores
_LANES = _INFO.num_lanes
_NW = _NC * _NS
_BPW = _B // _NW
_CH = 16
_NCH = _BPW // _CH
_CGROUPS = _CH // _LANES
_C = 128


def _mf_kernel(users_hbm, items_hbm, ue3_hbm, ie3_hbm, ub_hbm, ib_hbm,
               out_hbm, idx_u, idx_i, ublk, iblk, ublk2, iblk2,
               ub_v, ib_v, out_v, sem_u, sem_i, sem_u2, sem_i2,
               sem_ub, sem_ib):
    wid = lax.axis_index("s") * _NC + lax.axis_index("c")
    base = wid * _BPW

    pltpu.sync_copy(users_hbm.at[pl.ds(base, _BPW)], idx_u)
    pltpu.sync_copy(items_hbm.at[pl.ds(base, _BPW)], idx_i)

    bias_copies = []
    for c in range(_BPW // _C):
        s = pl.ds(c * _C, _C)
        bias_copies.append(pltpu.async_copy(
            ub_hbm.at[idx_u.at[s]], ub_v.at[s], sem_ub))
        bias_copies.append(pltpu.async_copy(
            ib_hbm.at[idx_i.at[s]], ib_v.at[s], sem_ib))

    def fire(c, ub, ib, su, si):
        def fire_body(g, carry2):
            s = pl.ds(c * _CH + g * _LANES, _LANES)
            vq_u = lax.shift_right_logical(idx_u[s], 3)
            vq_i = lax.shift_right_logical(idx_i[s], 3)
            for k in range(_LANES):
                r = g * _LANES + k
                pltpu.async_copy(ue3_hbm.at[vq_u[k]], ub.at[r], su)
                pltpu.async_copy(ie3_hbm.at[vq_i[k]], ib.at[r], si)
            return carry2

        lax.fori_loop(0, _CGROUPS, fire_body, 0)

    def drain(ub, ib, su, si):
        pltpu.make_async_copy(ue3_hbm.at[pl.ds(0, _CH)], ub, su).wait()
        pltpu.make_async_copy(ie3_hbm.at[pl.ds(0, _CH)], ib, si).wait()

    def compute(c, ub, ib):
        def group_body(g, carry3):
            sl = pl.ds(c * _CH + g * _LANES, _LANES)
            rloc = g * _LANES + lax.iota(jnp.int32, _LANES)
            usub = jnp.bitwise_and(idx_u[sl], 7)
            isub = jnp.bitwise_and(idx_i[sl], 7)
            acc = jnp.zeros((_LANES,), jnp.float32)
            for j in range(_L):
                jv = jnp.full((_LANES,), j, jnp.int32)
                u = plsc.load_gather(ub, [rloc, usub, jv])
                v = plsc.load_gather(ib, [rloc, isub, jv])
                acc = acc + u * v
            out_v[sl] = acc + _GLOBAL_BIAS
            return carry3

        lax.fori_loop(0, _CGROUPS, group_body, 0)

    fire(0, ublk, iblk, sem_u, sem_i)

    def pipe_body(t, carry):
        c0 = 2 * t
        fire(c0 + 1, ublk2, iblk2, sem_u2, sem_i2)
        drain(ublk, iblk, sem_u, sem_i)
        compute(c0, ublk, iblk)

        @pl.when(c0 + 2 < _NCH)
        def _():
            fire(c0 + 2, ublk, iblk, sem_u, sem_i)

        drain(ublk2, iblk2, sem_u2, sem_i2)
        compute(c0 + 1, ublk2, iblk2)
        return carry

    lax.fori_loop(0, _NCH // 2, pipe_body, 0)

    for cp in bias_copies:
        cp.wait()

    def bias_body(g, carry):
        s = pl.ds(g * _LANES, _LANES)
        out_v[s] = out_v[s] + ub_v[s] + ib_v[s]
        return carry

    lax.fori_loop(0, _BPW // _LANES, bias_body, 0)

    pltpu.sync_copy(out_v, out_hbm.at[pl.ds(base, _BPW)])


@functools.partial(
    pl.kernel,
    mesh=plsc.VectorSubcoreMesh(core_axis_name="c", subcore_axis_name="s"),
    out_type=jax.ShapeDtypeStruct((_B,), jnp.float32),
    compiler_params=pltpu.CompilerParams(needs_layout_passes=False),
    scratch_types=[
        pltpu.VMEM((_BPW,), jnp.int32),
        pltpu.VMEM((_BPW,), jnp.int32),
        pltpu.VMEM((_CH, 8, _L), jnp.float32),
        pltpu.VMEM((_CH, 8, _L), jnp.float32),
        pltpu.VMEM((_CH, 8, _L), jnp.float32),
        pltpu.VMEM((_CH, 8, _L), jnp.float32),
        pltpu.VMEM((_BPW,), jnp.float32),
        pltpu.VMEM((_BPW,), jnp.float32),
        pltpu.VMEM((_BPW,), jnp.float32),
        pltpu.SemaphoreType.DMA,
        pltpu.SemaphoreType.DMA,
        pltpu.SemaphoreType.DMA,
        pltpu.SemaphoreType.DMA,
        pltpu.SemaphoreType.DMA,
        pltpu.SemaphoreType.DMA,
    ],
)
def _mf_sc(users, items, ue3, ie3, ub, ib, out, *scratch):
    _mf_kernel(users, items, ue3, ie3, ub, ib, out, *scratch)


def kernel(users, items, uEmbed, itemEmbed, uBias, itemBias):
    n_u, l = uEmbed.shape
    n_i, _ = itemEmbed.shape
    score = _mf_sc(users.astype(jnp.int32), items.astype(jnp.int32),
                   uEmbed.reshape(n_u // 8, 8, l),
                   itemEmbed.reshape(n_i // 8, 8, l),
                   uBias.reshape(-1), itemBias.reshape(-1))
    return score.reshape(_B, 1)
